# baseline (speedup 1.0000x reference)
import functools

import jax
import jax.numpy as jnp
import numpy as np
from jax import lax
from jax.experimental import pallas as pl
from jax.experimental.pallas import tpu as pltpu
from jax.experimental.pallas import tpu_sc as plsc

N = 10000
D = 128
E = 320000
NC = 2
NS = 16
NW = NC * NS
EPW = E // NW
CH = 80
NCH = EPW // CH
G = 8
NG = NCH // G
NP = 10240
RPT = NP // NS

_PERM = np.empty(D, dtype=np.int32)
for _j in range(D // 32):
    for _t in range(16):
        _PERM[32 * _j + 2 * _t] = 32 * _j + _t
        _PERM[32 * _j + 2 * _t + 1] = 32 * _j + 16 + _t


def _tc1_body(x_ref, w_ref, al_ref, ar_ref, h_ref, el_ref, er_ref):
    h = lax.dot_general(
        x_ref[...], w_ref[...], (((1,), (0,)), ((), ())),
        preferred_element_type=jnp.float32,
    )
    h_ref[...] = h.astype(jnp.bfloat16)
    el_ref[...] = jnp.sum(h * al_ref[...], axis=1, keepdims=True)
    er_ref[...] = jnp.sum(h * ar_ref[...], axis=1, keepdims=True)


_BN = 1000

_tc1 = pl.pallas_call(
    _tc1_body,
    grid=(N // _BN,),
    in_specs=[
        pl.BlockSpec((_BN, D), lambda i: (i, 0)),
        pl.BlockSpec((D, D), lambda i: (0, 0)),
        pl.BlockSpec((1, D), lambda i: (0, 0)),
        pl.BlockSpec((1, D), lambda i: (0, 0)),
    ],
    out_specs=[
        pl.BlockSpec((_BN, D), lambda i: (i, 0)),
        pl.BlockSpec((_BN, 1), lambda i: (i, 0)),
        pl.BlockSpec((_BN, 1), lambda i: (i, 0)),
    ],
    out_shape=[
        jax.ShapeDtypeStruct((N, D), jnp.bfloat16),
        jax.ShapeDtypeStruct((N, 1), jnp.float32),
        jax.ShapeDtypeStruct((N, 1), jnp.float32),
    ],
)


_mesh = plsc.VectorSubcoreMesh(
    core_axis_name="c", subcore_axis_name="s", num_cores=NC, num_subcores=NS
)

_HIMASK = -65536


@functools.partial(
    pl.kernel,
    out_type=[
        jax.ShapeDtypeStruct((NC, NP, D), jnp.float32),
        jax.ShapeDtypeStruct((NC, NP, 16), jnp.float32),
    ],
    mesh=_mesh,
    compiler_params=pltpu.CompilerParams(
        needs_layout_passes=False, use_tc_tiling_on_sc=False
    ),
    scratch_types=[
        pltpu.VMEM((NCH, CH), jnp.int32),
        pltpu.VMEM((G, CH), jnp.int32),
        [pltpu.VMEM((CH,), jnp.float32) for _ in range(2)],
        [pltpu.VMEM((CH,), jnp.float32) for _ in range(2)],
        [pltpu.VMEM((CH,), jnp.float32) for _ in range(2)],
        [pltpu.VMEM((CH, D // 2), jnp.int32) for _ in range(2)],
        [pltpu.VMEM((CH, D), jnp.float32) for _ in range(2)],
        pltpu.VMEM((RPT,), jnp.float32),
        pltpu.VMEM((CH, 16), jnp.float32),
        pltpu.VMEM_SHARED((NP, D), jnp.float32),
        pltpu.VMEM_SHARED((NP,), jnp.float32),
        pltpu.SemaphoreType.DMA,
        pltpu.SemaphoreType.DMA,
    ],
)
def _sc_edges(src_hbm, dst_hbm, el_hbm, er_hbm, h_hbm, acc_out, den_out,
              sidx, didx, wv, elg, erg, hb, hf, dst_v, dre_v,
              acc_sh, den_sh, sem_g, sem_s):
    cid = lax.axis_index("c")
    sid = lax.axis_index("s")
    wid = cid * NS + sid
    zero16 = jnp.zeros((16,), jnp.float32)
    lane = lax.iota(jnp.int32, 16)

    def _zrow(i, _):
        for j in range(D // 16):
            hf[0][i, pl.ds(j * 16, 16)] = zero16
        return 0

    lax.fori_loop(0, CH, _zrow, 0)
    for j in range(CH // 16):
        wv[0][pl.ds(j * 16, 16)] = zero16
    r0 = sid * RPT
    for b in range(RPT // CH):
        pltpu.sync_copy(hf[0], acc_sh.at[pl.ds(r0 + b * CH, CH)])
        pltpu.sync_copy(wv[0], den_sh.at[pl.ds(r0 + b * CH, CH)])
    plsc.subcore_barrier()

    pltpu.sync_copy(src_hbm.at[wid], sidx)

    def _gathers(k, o, b):
        return (
            pltpu.async_copy(h_hbm.at[sidx.at[k]], hb[b], sem_g),
            pltpu.async_copy(el_hbm.at[sidx.at[k]], elg[b], sem_g),
            pltpu.async_copy(er_hbm.at[didx.at[o]], erg[b], sem_g),
        )

    def _compute_scale(b):
        for j in range(CH // 16):
            e = elg[b][pl.ds(j * 16, 16)] + erg[b][pl.ds(j * 16, 16)]
            e = jnp.maximum(e, 0.2 * e)
            wv[b][pl.ds(j * 16, 16)] = jnp.exp(e)

        def _row(i, _):
            wi = plsc.load_gather(wv[b], [lane * 0 + i])
            for j in range(D // 32):
                x = hb[b][i, pl.ds(j * 16, 16)]
                lo = plsc.bitcast(x << 16, jnp.float32)
                hi = plsc.bitcast(x & _HIMASK, jnp.float32)
                hf[b][i, pl.ds(j * 32, 16)] = lo * wi
                hf[b][i, pl.ds(j * 32 + 16, 16)] = hi * wi
            return 0

        lax.fori_loop(0, CH, _row, 0, unroll=4)

    def _group(gi, n):
        k0 = gi * G
        pltpu.sync_copy(dst_hbm.at[wid, pl.ds(k0, n)], didx.at[pl.ds(0, n)])
        d = [None] * n
        s = [None] * n
        d[0] = _gathers(k0, 0, 0)
        for o in range(n):
            b = o % 2
            if o >= 1:
                s[o - 1][0].wait()
                s[o - 1][1].wait()
            if o + 1 < n:
                d[o + 1] = _gathers(k0 + o + 1, o + 1, 1 - b)
            for h in d[o]:
                h.wait()
            _compute_scale(b)
            s[o] = (
                pltpu.async_copy(
                    hf[b], acc_sh.at[didx.at[o]], sem_s, add=True
                ),
                pltpu.async_copy(
                    wv[b], den_sh.at[didx.at[o]], sem_s, add=True
                ),
            )
        s[n - 1][0].wait()
        s[n - 1][1].wait()

    @pl.loop(0, NG)
    def _steady(gi):
        _group(gi, G)

    _group(NG, NCH - NG * G)
    plsc.subcore_barrier()

    for b in range(RPT // CH):
        rr = r0 + b * CH
        pltpu.sync_copy(acc_sh.at[pl.ds(rr, CH)], hf[0])
        pltpu.sync_copy(hf[0], acc_out.at[cid, pl.ds(rr, CH)])
    pltpu.sync_copy(den_sh.at[pl.ds(r0, RPT)], dst_v)
    for b in range(RPT // CH):
        def _dex(g, _):
            dg = dst_v[pl.ds(b * CH + g * 16, 16)]
            plsc.store_scatter(dre_v, [g * 16 + lane, lane * 0], dg)
            return 0

        lax.fori_loop(0, CH // 16, _dex, 0)
        pltpu.sync_copy(dre_v, den_out.at[cid, pl.ds(r0 + b * CH, CH)])


def _tc2_body(a0_ref, a1_ref, d0_ref, d1_ref, o_ref):
    s = a0_ref[0] + a1_ref[0]
    d = d0_ref[0, :, 0:1] + d1_ref[0, :, 0:1]
    o_ref[...] = s / (d + 1e-9)


_tc2 = pl.pallas_call(
    _tc2_body,
    grid=(N // _BN,),
    in_specs=[
        pl.BlockSpec((1, _BN, D), lambda i: (0, i, 0)),
        pl.BlockSpec((1, _BN, D), lambda i: (1, i, 0)),
        pl.BlockSpec((1, _BN, 16), lambda i: (0, i, 0)),
        pl.BlockSpec((1, _BN, 16), lambda i: (1, i, 0)),
    ],
    out_specs=pl.BlockSpec((_BN, D), lambda i: (i, 0)),
    out_shape=jax.ShapeDtypeStruct((N, D), jnp.float32),
)


def kernel(features, edge_index, W, attn_l, attn_r):
    perm = jnp.asarray(_PERM)
    h_bf, el2, er2 = _tc1(features, W[:, perm], attn_l[:, perm], attn_r[:, perm])
    el = el2.reshape(N)
    er = er2.reshape(N)
    h_i32 = lax.bitcast_convert_type(h_bf.reshape(N, D // 2, 2), jnp.int32)
    src = edge_index[0].reshape(NW, NCH, CH)
    dst = edge_index[1].reshape(NW, NCH, CH)
    acc, den = _sc_edges(src, dst, el, er, h_i32)
    return _tc2(acc, acc, den, den)


# speedup vs baseline: 1.6695x; 1.6695x over previous
import functools

import jax
import jax.numpy as jnp
from jax import lax
from jax.experimental import pallas as pl
from jax.experimental.pallas import tpu as pltpu
from jax.experimental.pallas import tpu_sc as plsc

N = 10000
D = 128
E = 320000
NC = 2
NS = 16
NW = NC * NS
EPW = E // NW
CH = 80
NCH = EPW // CH
G = 8
NG = NCH // G
NP = 10240
RPT = NP // NS
HS = CH // 2


def _tc1_body(x_ref, w_ref, al_ref, ar_ref, h_ref, el_ref, er_ref):
    h = lax.dot_general(
        x_ref[...], w_ref[...], (((1,), (0,)), ((), ())),
        preferred_element_type=jnp.float32,
    )
    h_ref[...] = h
    el_ref[...] = jnp.sum(h * al_ref[...], axis=1, keepdims=True)
    er_ref[...] = jnp.sum(h * ar_ref[...], axis=1, keepdims=True)


_BN = 1000

_tc1 = pl.pallas_call(
    _tc1_body,
    grid=(N // _BN,),
    in_specs=[
        pl.BlockSpec((_BN, D), lambda i: (i, 0)),
        pl.BlockSpec((D, D), lambda i: (0, 0)),
        pl.BlockSpec((1, D), lambda i: (0, 0)),
        pl.BlockSpec((1, D), lambda i: (0, 0)),
    ],
    out_specs=[
        pl.BlockSpec((_BN, D), lambda i: (i, 0)),
        pl.BlockSpec((_BN, 1), lambda i: (i, 0)),
        pl.BlockSpec((_BN, 1), lambda i: (i, 0)),
    ],
    out_shape=[
        jax.ShapeDtypeStruct((N, D), jnp.float32),
        jax.ShapeDtypeStruct((N, 1), jnp.float32),
        jax.ShapeDtypeStruct((N, 1), jnp.float32),
    ],
)


_mesh = plsc.VectorSubcoreMesh(
    core_axis_name="c", subcore_axis_name="s", num_cores=NC, num_subcores=NS
)


@functools.partial(
    pl.kernel,
    out_type=[
        jax.ShapeDtypeStruct((NC, NP, D), jnp.float32),
        jax.ShapeDtypeStruct((NC, NP, 16), jnp.float32),
    ],
    mesh=_mesh,
    compiler_params=pltpu.CompilerParams(
        needs_layout_passes=False, use_tc_tiling_on_sc=False
    ),
    scratch_types=[
        pltpu.VMEM((NCH, CH), jnp.int32),
        pltpu.VMEM((NCH, CH), jnp.int32),
        [pltpu.VMEM((CH,), jnp.float32) for _ in range(2)],
        [pltpu.VMEM((CH,), jnp.float32) for _ in range(2)],
        [pltpu.VMEM((CH,), jnp.float32) for _ in range(2)],
        [pltpu.VMEM((CH, D), jnp.float32) for _ in range(2)],
        pltpu.VMEM((RPT,), jnp.float32),
        pltpu.VMEM((CH, 16), jnp.float32),
        pltpu.VMEM_SHARED((NP, D), jnp.float32),
        pltpu.VMEM_SHARED((NP,), jnp.float32),
        pltpu.SemaphoreType.DMA,
        pltpu.SemaphoreType.DMA,
    ],
)
def _sc_edges(src_hbm, dst_hbm, el_hbm, er_hbm, h_hbm, acc_out, den_out,
              sidx, didx, wv, elg, erg, hb, dst_v, dre_v,
              acc_sh, den_sh, sem_g, sem_s):
    cid = lax.axis_index("c")
    sid = lax.axis_index("s")
    wid = cid * NS + sid
    zero16 = jnp.zeros((16,), jnp.float32)
    lane = lax.iota(jnp.int32, 16)

    def _zrow(i, _):
        for j in range(D // 16):
            hb[0][i, pl.ds(j * 16, 16)] = zero16
        return 0

    lax.fori_loop(0, CH, _zrow, 0)
    for j in range(CH // 16):
        wv[0][pl.ds(j * 16, 16)] = zero16
    r0 = sid * RPT
    for b in range(RPT // CH):
        pltpu.sync_copy(hb[0], acc_sh.at[pl.ds(r0 + b * CH, CH)])
        pltpu.sync_copy(wv[0], den_sh.at[pl.ds(r0 + b * CH, CH)])
    plsc.subcore_barrier()

    pltpu.sync_copy(src_hbm.at[wid], sidx)
    pltpu.sync_copy(dst_hbm.at[wid], didx)

    def _gathers(k, b):
        return (
            pltpu.async_copy(
                h_hbm.at[sidx.at[k, pl.ds(0, HS)]], hb[b].at[pl.ds(0, HS)],
                sem_g,
            ),
            pltpu.async_copy(
                h_hbm.at[sidx.at[k, pl.ds(HS, HS)]], hb[b].at[pl.ds(HS, HS)],
                sem_g,
            ),
            pltpu.async_copy(el_hbm.at[sidx.at[k]], elg[b], sem_g),
            pltpu.async_copy(er_hbm.at[didx.at[k]], erg[b], sem_g),
        )

    def _compute_scale(b):
        for j in range(CH // 16):
            e = elg[b][pl.ds(j * 16, 16)] + erg[b][pl.ds(j * 16, 16)]
            e = jnp.maximum(e, 0.2 * e)
            wv[b][pl.ds(j * 16, 16)] = jnp.exp(e)

        def _row(i, _):
            wi = plsc.load_gather(wv[b], [lane * 0 + i])
            for j in range(D // 16):
                hb[b][i, pl.ds(j * 16, 16)] = hb[b][i, pl.ds(j * 16, 16)] * wi
            return 0

        lax.fori_loop(0, CH, _row, 0, unroll=4)

    def _group(gi, n):
        k0 = gi * G
        d = [None] * n
        s = [None] * n
        d[0] = _gathers(k0, 0)
        for o in range(n):
            b = o % 2
            if o >= 1:
                s[o - 1][0].wait()
                s[o - 1][1].wait()
            if o + 1 < n:
                d[o + 1] = _gathers(k0 + o + 1, 1 - b)
            for h in d[o]:
                h.wait()
            _compute_scale(b)
            s[o] = (
                pltpu.async_copy(
                    hb[b], acc_sh.at[didx.at[k0 + o]], sem_s, add=True
                ),
                pltpu.async_copy(
                    wv[b], den_sh.at[didx.at[k0 + o]], sem_s, add=True
                ),
            )
        s[n - 1][0].wait()
        s[n - 1][1].wait()

    @pl.loop(0, NG)
    def _steady(gi):
        _group(gi, G)

    _group(NG, NCH - NG * G)
    plsc.subcore_barrier()

    for b in range(RPT // CH):
        rr = r0 + b * CH
        pltpu.sync_copy(acc_sh.at[pl.ds(rr, CH)], hb[0])
        pltpu.sync_copy(hb[0], acc_out.at[cid, pl.ds(rr, CH)])
    pltpu.sync_copy(den_sh.at[pl.ds(r0, RPT)], dst_v)
    for b in range(RPT // CH):
        def _dex(g, _):
            dg = dst_v[pl.ds(b * CH + g * 16, 16)]
            plsc.store_scatter(dre_v, [g * 16 + lane, lane * 0], dg)
            return 0

        lax.fori_loop(0, CH // 16, _dex, 0)
        pltpu.sync_copy(dre_v, den_out.at[cid, pl.ds(r0 + b * CH, CH)])


def _tc2_body(a0_ref, a1_ref, d0_ref, d1_ref, o_ref):
    s = a0_ref[0] + a1_ref[0]
    d = d0_ref[0, :, 0:1] + d1_ref[0, :, 0:1]
    o_ref[...] = s / (d + 1e-9)


_tc2 = pl.pallas_call(
    _tc2_body,
    grid=(N // _BN,),
    in_specs=[
        pl.BlockSpec((1, _BN, D), lambda i: (0, i, 0)),
        pl.BlockSpec((1, _BN, D), lambda i: (1, i, 0)),
        pl.BlockSpec((1, _BN, 16), lambda i: (0, i, 0)),
        pl.BlockSpec((1, _BN, 16), lambda i: (1, i, 0)),
    ],
    out_specs=pl.BlockSpec((_BN, D), lambda i: (i, 0)),
    out_shape=jax.ShapeDtypeStruct((N, D), jnp.float32),
)


def kernel(features, edge_index, W, attn_l, attn_r):
    h, el2, er2 = _tc1(features, W, attn_l, attn_r)
    el = el2.reshape(N)
    er = er2.reshape(N)
    src = edge_index[0].reshape(NW, NCH, CH)
    dst = edge_index[1].reshape(NW, NCH, CH)
    acc, den = _sc_edges(src, dst, el, er, h)
    return _tc2(acc, acc, den, den)
